# baseline (speedup 1.0000x reference)
import jax
import jax.numpy as jnp
from jax.experimental import pallas as pl
from jax.experimental.pallas import tpu as pltpu

N_ROWS = 1_000_000
DIM = 32
BLOCK = 4000


def _tc_body(x_ref, o_ref):
    x = x_ref[...]
    ss = jnp.sum(x * x, axis=1, keepdims=True)
    ss = jnp.maximum(ss, 1e-16)
    n = jnp.sqrt(ss)
    scale = (jnp.exp(n) - jnp.exp(-n)) * 0.5 / n
    scale = jnp.where(n < 1e-3, 1.0 + ss * (1.0 / 6.0), scale)
    o_ref[...] = x * scale


@jax.jit
def kernel(tangent_embeddings):
    return pl.pallas_call(
        _tc_body,
        out_shape=jax.ShapeDtypeStruct((N_ROWS, DIM), jnp.float32),
        grid=(N_ROWS // BLOCK,),
        in_specs=[pl.BlockSpec((BLOCK, DIM), lambda i: (i, 0))],
        out_specs=pl.BlockSpec((BLOCK, DIM), lambda i: (i, 0)),
        compiler_params=pltpu.CompilerParams(
            dimension_semantics=("arbitrary",)
        ),
    )(tangent_embeddings)
